# baseline (speedup 1.0000x reference)
import functools

import jax
import jax.numpy as jnp
from jax import lax
from jax.experimental import pallas as pl
from jax.experimental.pallas import tpu as pltpu
from jax.experimental.pallas import tpu_sc as plsc

_V = 1_000_000
_D = 64
_J = 200
_BB = 4096
_NW = 32
_BLK = _BB // _NW
_JB = 8
_NJB = _J // _JB
_L = 16

_CV = 256
_NFULL = (_V // 128 // 2) * 2 // 2 * 2 * 64 // 64
_NFULL = 999_936 // _CV
_TAILV = _V - _NFULL * _CV
_PAIRS_PER_IT = 2 * _NW
_NIT = -(-_NFULL // _PAIRS_PER_IT)


@functools.partial(
    pl.kernel,
    out_type=jax.ShapeDtypeStruct((_V // 2, 128), jnp.float32),
    mesh=plsc.VectorSubcoreMesh(core_axis_name="c", subcore_axis_name="s"),
    compiler_params=pltpu.CompilerParams(needs_layout_passes=False),
    scratch_types=[
        pltpu.VMEM((_D, _CV), jnp.float32),
        pltpu.VMEM((_D, _CV), jnp.float32),
        pltpu.VMEM((_CV // 2, 128), jnp.float32),
        pltpu.VMEM((_CV // 2, 128), jnp.float32),
        pltpu.VMEM((_D, _TAILV), jnp.float32),
        pltpu.SemaphoreType.DMA,
        pltpu.SemaphoreType.DMA,
    ],
)
def _relayout(tt_hbm, tail_hbm, out_hbm, in0, in1, out0, out1, tin, sem0, sem1):
    wid = lax.axis_index("s") * 2 + lax.axis_index("c")
    ivec = lax.iota(jnp.int32, _L)
    rowhalf = lax.shift_right_logical(ivec, 1)
    colbase = lax.shift_left(lax.bitwise_and(ivec, 1), 6)

    def transpose_chunk(in_v, out_v):
        def c16_body(c16, carry):
            rbase = rowhalf + c16 * (_L // 2)
            for d in range(_D):
                v = in_v[d, pl.ds(c16 * _L, _L)]
                plsc.store_scatter(out_v, [rbase, colbase + d], v)
            return carry

        lax.fori_loop(0, _CV // _L, c16_body, 0)

    def body(t, carry):
        g0 = wid + _PAIRS_PER_IT * t
        g1 = wid + _NW + _PAIRS_PER_IT * t
        c0 = jnp.where(g0 < _NFULL, g0 * _CV, 0)
        c1 = jnp.where(g1 < _NFULL, g1 * _CV, 0)
        c0 = pl.multiple_of(c0, _CV)
        c1 = pl.multiple_of(c1, _CV)
        h0 = pltpu.async_copy(tt_hbm.at[:, pl.ds(c0, _CV)], in0, sem0)
        h1 = pltpu.async_copy(tt_hbm.at[:, pl.ds(c1, _CV)], in1, sem1)
        h0.wait()
        transpose_chunk(in0, out0)
        s0 = pltpu.async_copy(
            out0, out_hbm.at[pl.ds(pl.multiple_of(c0 // 2, _CV // 2), _CV // 2)],
            sem0)
        h1.wait()
        transpose_chunk(in1, out1)
        s1 = pltpu.async_copy(
            out1, out_hbm.at[pl.ds(pl.multiple_of(c1 // 2, _CV // 2), _CV // 2)],
            sem1)
        s0.wait()
        s1.wait()
        return carry

    lax.fori_loop(0, _NIT, body, 0)

    @pl.when(wid == 0)
    def _tail():
        pltpu.sync_copy(tail_hbm, tin)
        def c16_body(c16, carry):
            rbase = rowhalf + c16 * (_L // 2)
            for d in range(_D):
                v = tin[d, pl.ds(c16 * _L, _L)]
                plsc.store_scatter(out0, [rbase, colbase + d], v)
            return carry
        lax.fori_loop(0, _TAILV // _L, c16_body, 0)
        pltpu.sync_copy(
            out0.at[pl.ds(0, _TAILV // 2)],
            out_hbm.at[pl.ds(_NFULL * _CV // 2, _TAILV // 2)],
        )


@functools.partial(
    pl.kernel,
    out_type=jax.ShapeDtypeStruct((_J * _D, _BB), jnp.float32),
    mesh=plsc.VectorSubcoreMesh(core_axis_name="c", subcore_axis_name="s"),
    compiler_params=pltpu.CompilerParams(needs_layout_passes=False),
    scratch_types=[
        pltpu.VMEM((_JB, _BLK), jnp.int32),
        pltpu.VMEM((_JB, _BLK), jnp.int32),
        pltpu.VMEM((_JB, _BLK), jnp.int32),
        pltpu.VMEM((_BLK, 128), jnp.float32),
        pltpu.VMEM((_BLK, 128), jnp.float32),
        pltpu.VMEM((_D, _BLK), jnp.float32),
        pltpu.VMEM((_D, _BLK), jnp.float32),
        pltpu.SemaphoreType.DMA,
        pltpu.SemaphoreType.DMA,
        pltpu.SemaphoreType.DMA,
        pltpu.SemaphoreType.DMA,
    ],
)
def _embed_gather(xt_hbm, table_hbm, out_hbm, idx_v, half_v, par_v,
                  rows0, rows1, ot0, ot1, g0, g1, o0, o1):
    wid = lax.axis_index("s") * 2 + lax.axis_index("c")
    col0 = pl.multiple_of(wid * _BLK, _BLK)
    ivec = lax.iota(jnp.int32, _L)
    zvec = ivec * 0
    rvecs = [ivec + k * _L for k in range(_D // _L)]

    def transpose_block(rows_v, jj, ot_v):
        parvs = [par_v[jj, pl.ds(i * _L, _L)] for i in range(_BLK // _L)]
        rrows = [ivec + i * _L for i in range(_BLK // _L)]

        for d in range(_D):
            for i in range(_BLK // _L):
                v = plsc.load_gather(rows_v, [rrows[i], parvs[i] + d])
                ot_v[d, pl.ds(i * _L, _L)] = v

    def do_pair(j0, jj, rows_a, rows_b, ot_a, ot_b, sga, sgb, soa, sob):
        ha = pltpu.async_copy(table_hbm.at[half_v.at[jj]], rows_a, sga)
        hb = pltpu.async_copy(table_hbm.at[half_v.at[jj + 1]], rows_b, sgb)
        ha.wait()
        transpose_block(rows_a, jj, ot_a)
        row_a = pl.multiple_of((j0 + jj) * _D, _D)
        sa = pltpu.async_copy(
            ot_a, out_hbm.at[pl.ds(row_a, _D), pl.ds(col0, _BLK)], soa)
        hb.wait()
        transpose_block(rows_b, jj + 1, ot_b)
        row_b = pl.multiple_of((j0 + jj + 1) * _D, _D)
        sb = pltpu.async_copy(
            ot_b, out_hbm.at[pl.ds(row_b, _D), pl.ds(col0, _BLK)], sob)
        sa.wait()
        sb.wait()

    def step(ja, carry):
        j0 = pl.multiple_of(ja * _JB, _JB)
        pltpu.sync_copy(xt_hbm.at[pl.ds(j0, _JB), pl.ds(col0, _BLK)], idx_v)
        for jj in range(_JB):
            for i in range(_BLK // _L):
                v = idx_v[jj, pl.ds(i * _L, _L)]
                half_v[jj, pl.ds(i * _L, _L)] = lax.shift_right_logical(v, 1)
                par_v[jj, pl.ds(i * _L, _L)] = lax.shift_left(
                    lax.bitwise_and(v, 1), 6)

        def pair_body(p, carry2):
            do_pair(j0, p * 2, rows0, rows1, ot0, ot1, g0, g1, o0, o1)
            return carry2

        lax.fori_loop(0, _JB // 2, pair_body, 0)
        return carry

    lax.fori_loop(0, _NJB, step, 0)


def kernel(x, table):
    tt = table.T
    tail_t = table[_NFULL * _CV:].T
    tr = _relayout(tt, tail_t)
    xt = x.T
    out = _embed_gather(xt, tr)
    return out.reshape(_J, _D, _BB).transpose(2, 0, 1)


# speedup vs baseline: 1.2558x; 1.2558x over previous
import functools

import jax
import jax.numpy as jnp
from jax import lax
from jax.experimental import pallas as pl
from jax.experimental.pallas import tpu as pltpu
from jax.experimental.pallas import tpu_sc as plsc

_V = 1_000_000
_D = 64
_J = 200
_BB = 4096
_NW = 32
_BLK = _BB // _NW
_JB = 8
_NJB = _J // _JB
_L = 16

_CV = 256
_NFULL = (_V // 128 // 2) * 2 // 2 * 2 * 64 // 64
_NFULL = 999_936 // _CV
_TAILV = _V - _NFULL * _CV
_PAIRS_PER_IT = 2 * _NW
_NIT = -(-_NFULL // _PAIRS_PER_IT)


@functools.partial(
    pl.kernel,
    out_type=jax.ShapeDtypeStruct((_V // 2, 128), jnp.float32),
    mesh=plsc.VectorSubcoreMesh(core_axis_name="c", subcore_axis_name="s"),
    compiler_params=pltpu.CompilerParams(needs_layout_passes=False),
    scratch_types=[
        pltpu.VMEM((_D, _CV), jnp.float32),
        pltpu.VMEM((_D, _CV), jnp.float32),
        pltpu.VMEM((_CV // 2, 128), jnp.float32),
        pltpu.VMEM((_CV // 2, 128), jnp.float32),
        pltpu.VMEM((_D, _TAILV), jnp.float32),
        pltpu.SemaphoreType.DMA,
        pltpu.SemaphoreType.DMA,
    ],
)
def _relayout(tt_hbm, tail_hbm, out_hbm, in0, in1, out0, out1, tin, sem0, sem1):
    wid = lax.axis_index("s") * 2 + lax.axis_index("c")
    ivec = lax.iota(jnp.int32, _L)
    rowhalf = lax.shift_right_logical(ivec, 1)
    colbase = lax.shift_left(lax.bitwise_and(ivec, 1), 6)

    def transpose_chunk(in_v, out_v):
        def c16_body(c16, carry):
            rbase = rowhalf + c16 * (_L // 2)
            for dg in range(_D // 8):
                vs = [in_v[dg * 8 + t, pl.ds(c16 * _L, _L)] for t in range(8)]
                for t in range(8):
                    plsc.store_scatter(
                        out_v, [rbase, colbase + (dg * 8 + t)], vs[t])
            return carry

        lax.fori_loop(0, _CV // _L, c16_body, 0)

    def body(t, carry):
        g0 = wid + _PAIRS_PER_IT * t
        g1 = wid + _NW + _PAIRS_PER_IT * t
        c0 = jnp.where(g0 < _NFULL, g0 * _CV, 0)
        c1 = jnp.where(g1 < _NFULL, g1 * _CV, 0)
        c0 = pl.multiple_of(c0, _CV)
        c1 = pl.multiple_of(c1, _CV)
        h0 = pltpu.async_copy(tt_hbm.at[:, pl.ds(c0, _CV)], in0, sem0)
        h1 = pltpu.async_copy(tt_hbm.at[:, pl.ds(c1, _CV)], in1, sem1)
        h0.wait()
        transpose_chunk(in0, out0)
        s0 = pltpu.async_copy(
            out0, out_hbm.at[pl.ds(pl.multiple_of(c0 // 2, _CV // 2), _CV // 2)],
            sem0)
        h1.wait()
        transpose_chunk(in1, out1)
        s1 = pltpu.async_copy(
            out1, out_hbm.at[pl.ds(pl.multiple_of(c1 // 2, _CV // 2), _CV // 2)],
            sem1)
        s0.wait()
        s1.wait()
        return carry

    lax.fori_loop(0, _NIT, body, 0)

    @pl.when(wid == 0)
    def _tail():
        pltpu.sync_copy(tail_hbm, tin)
        def c16_body(c16, carry):
            rbase = rowhalf + c16 * (_L // 2)
            for d in range(_D):
                v = tin[d, pl.ds(c16 * _L, _L)]
                plsc.store_scatter(out0, [rbase, colbase + d], v)
            return carry
        lax.fori_loop(0, _TAILV // _L, c16_body, 0)
        pltpu.sync_copy(
            out0.at[pl.ds(0, _TAILV // 2)],
            out_hbm.at[pl.ds(_NFULL * _CV // 2, _TAILV // 2)],
        )


@functools.partial(
    pl.kernel,
    out_type=jax.ShapeDtypeStruct((_J * _D, _BB), jnp.float32),
    mesh=plsc.VectorSubcoreMesh(core_axis_name="c", subcore_axis_name="s"),
    compiler_params=pltpu.CompilerParams(needs_layout_passes=False),
    scratch_types=[
        pltpu.VMEM((_JB, _BLK), jnp.int32),
        pltpu.VMEM((_JB, _BLK), jnp.int32),
        pltpu.VMEM((_JB, _BLK), jnp.int32),
        pltpu.VMEM((_BLK, 128), jnp.float32),
        pltpu.VMEM((_BLK, 128), jnp.float32),
        pltpu.VMEM((_D, _BLK), jnp.float32),
        pltpu.VMEM((_D, _BLK), jnp.float32),
        pltpu.SemaphoreType.DMA,
        pltpu.SemaphoreType.DMA,
        pltpu.SemaphoreType.DMA,
        pltpu.SemaphoreType.DMA,
    ],
)
def _embed_gather(xt_hbm, table_hbm, out_hbm, idx_v, half_v, par_v,
                  rows0, rows1, ot0, ot1, g0, g1, o0, o1):
    wid = lax.axis_index("s") * 2 + lax.axis_index("c")
    col0 = pl.multiple_of(wid * _BLK, _BLK)
    ivec = lax.iota(jnp.int32, _L)
    zvec = ivec * 0
    rvecs = [ivec + k * _L for k in range(_D // _L)]

    def transpose_block(rows_v, jj, ot_v):
        parvs = [par_v[jj, pl.ds(i * _L, _L)] for i in range(_BLK // _L)]
        rrows = [ivec + i * _L for i in range(_BLK // _L)]

        for d in range(_D):
            vs = [
                plsc.load_gather(rows_v, [rrows[i], parvs[i] + d])
                for i in range(_BLK // _L)
            ]
            for i in range(_BLK // _L):
                ot_v[d, pl.ds(i * _L, _L)] = vs[i]

    def do_pair(j0, jj, rows_a, rows_b, ot_a, ot_b, sga, sgb, soa, sob):
        ha = pltpu.async_copy(table_hbm.at[half_v.at[jj]], rows_a, sga)
        hb = pltpu.async_copy(table_hbm.at[half_v.at[jj + 1]], rows_b, sgb)
        ha.wait()
        transpose_block(rows_a, jj, ot_a)
        row_a = pl.multiple_of((j0 + jj) * _D, _D)
        sa = pltpu.async_copy(
            ot_a, out_hbm.at[pl.ds(row_a, _D), pl.ds(col0, _BLK)], soa)
        hb.wait()
        transpose_block(rows_b, jj + 1, ot_b)
        row_b = pl.multiple_of((j0 + jj + 1) * _D, _D)
        sb = pltpu.async_copy(
            ot_b, out_hbm.at[pl.ds(row_b, _D), pl.ds(col0, _BLK)], sob)
        sa.wait()
        sb.wait()

    def step(ja, carry):
        j0 = pl.multiple_of(ja * _JB, _JB)
        pltpu.sync_copy(xt_hbm.at[pl.ds(j0, _JB), pl.ds(col0, _BLK)], idx_v)
        for jj in range(_JB):
            for i in range(_BLK // _L):
                v = idx_v[jj, pl.ds(i * _L, _L)]
                half_v[jj, pl.ds(i * _L, _L)] = lax.shift_right_logical(v, 1)
                par_v[jj, pl.ds(i * _L, _L)] = lax.shift_left(
                    lax.bitwise_and(v, 1), 6)

        def pair_body(p, carry2):
            do_pair(j0, p * 2, rows0, rows1, ot0, ot1, g0, g1, o0, o1)
            return carry2

        lax.fori_loop(0, _JB // 2, pair_body, 0)
        return carry

    lax.fori_loop(0, _NJB, step, 0)


def kernel(x, table):
    tt = table.T
    tail_t = table[_NFULL * _CV:].T
    tr = _relayout(tt, tail_t)
    xt = x.T
    out = _embed_gather(xt, tr)
    return out.reshape(_J, _D, _BB).transpose(2, 0, 1)


# speedup vs baseline: 1.3696x; 1.0906x over previous
import functools

import jax
import jax.numpy as jnp
from jax import lax
from jax.experimental import pallas as pl
from jax.experimental.pallas import tpu as pltpu
from jax.experimental.pallas import tpu_sc as plsc

_V = 1_000_000
_D = 64
_J = 200
_BB = 4096
_NW = 32
_BLK = _BB // _NW
_JB = 8
_NJB = _J // _JB
_L = 16

_CV = 256
_NFULL = (_V // 128 // 2) * 2 // 2 * 2 * 64 // 64
_NFULL = 999_936 // _CV
_TAILV = _V - _NFULL * _CV
_PAIRS_PER_IT = 2 * _NW
_NIT = -(-_NFULL // _PAIRS_PER_IT)


@functools.partial(
    pl.kernel,
    out_type=jax.ShapeDtypeStruct((_V // 2, 128), jnp.float32),
    mesh=plsc.VectorSubcoreMesh(core_axis_name="c", subcore_axis_name="s"),
    compiler_params=pltpu.CompilerParams(needs_layout_passes=False),
    scratch_types=[
        pltpu.VMEM((_D, _CV), jnp.float32),
        pltpu.VMEM((_D, _CV), jnp.float32),
        pltpu.VMEM((_CV // 2, 128), jnp.float32),
        pltpu.VMEM((_CV // 2, 128), jnp.float32),
        pltpu.VMEM((_D, _TAILV), jnp.float32),
        pltpu.SemaphoreType.DMA,
        pltpu.SemaphoreType.DMA,
    ],
)
def _relayout(tt_hbm, tail_hbm, out_hbm, in0, in1, out0, out1, tin, sem0, sem1):
    wid = lax.axis_index("s") * 2 + lax.axis_index("c")
    ivec = lax.iota(jnp.int32, _L)
    rowhalf = lax.shift_right_logical(ivec, 1)
    colbase = lax.shift_left(lax.bitwise_and(ivec, 1), 6)

    def transpose_chunk(in_v, out_v):
        def c16_body(c16, carry):
            rbase = rowhalf + c16 * (_L // 2)
            for dg in range(_D // 8):
                vs = [in_v[dg * 8 + t, pl.ds(c16 * _L, _L)] for t in range(8)]
                for t in range(8):
                    plsc.store_scatter(
                        out_v, [rbase, colbase + (dg * 8 + t)], vs[t])
            return carry

        lax.fori_loop(0, _CV // _L, c16_body, 0)

    def body(t, carry):
        g0 = wid + _PAIRS_PER_IT * t
        g1 = wid + _NW + _PAIRS_PER_IT * t
        c0 = jnp.where(g0 < _NFULL, g0 * _CV, 0)
        c1 = jnp.where(g1 < _NFULL, g1 * _CV, 0)
        c0 = pl.multiple_of(c0, _CV)
        c1 = pl.multiple_of(c1, _CV)
        h0 = pltpu.async_copy(tt_hbm.at[:, pl.ds(c0, _CV)], in0, sem0)
        h1 = pltpu.async_copy(tt_hbm.at[:, pl.ds(c1, _CV)], in1, sem1)
        h0.wait()
        transpose_chunk(in0, out0)
        s0 = pltpu.async_copy(
            out0, out_hbm.at[pl.ds(pl.multiple_of(c0 // 2, _CV // 2), _CV // 2)],
            sem0)
        h1.wait()
        transpose_chunk(in1, out1)
        s1 = pltpu.async_copy(
            out1, out_hbm.at[pl.ds(pl.multiple_of(c1 // 2, _CV // 2), _CV // 2)],
            sem1)
        s0.wait()
        s1.wait()
        return carry

    lax.fori_loop(0, _NIT, body, 0)

    @pl.when(wid == 0)
    def _tail():
        pltpu.sync_copy(tail_hbm, tin)
        def c16_body(c16, carry):
            rbase = rowhalf + c16 * (_L // 2)
            for d in range(_D):
                v = tin[d, pl.ds(c16 * _L, _L)]
                plsc.store_scatter(out0, [rbase, colbase + d], v)
            return carry
        lax.fori_loop(0, _TAILV // _L, c16_body, 0)
        pltpu.sync_copy(
            out0.at[pl.ds(0, _TAILV // 2)],
            out_hbm.at[pl.ds(_NFULL * _CV // 2, _TAILV // 2)],
        )


@functools.partial(
    pl.kernel,
    out_type=jax.ShapeDtypeStruct((_J * _D, _BB), jnp.float32),
    mesh=plsc.VectorSubcoreMesh(core_axis_name="c", subcore_axis_name="s"),
    compiler_params=pltpu.CompilerParams(needs_layout_passes=False),
    scratch_types=[
        pltpu.VMEM((_JB, _BLK), jnp.int32),
        pltpu.VMEM((_JB, _BLK), jnp.int32),
        pltpu.VMEM((_JB, _BLK), jnp.int32),
        pltpu.VMEM((4 * _BLK, 128), jnp.float32),
        pltpu.VMEM((_D, _BLK), jnp.float32),
        pltpu.VMEM((_D, _BLK), jnp.float32),
        pltpu.SemaphoreType.DMA,
        pltpu.SemaphoreType.DMA,
        pltpu.SemaphoreType.DMA,
    ],
)
def _embed_gather(xt_hbm, table_hbm, out_hbm, idx_v, half_v, par_v,
                  rows_big, ot0, ot1, gsem, o0, o1):
    wid = lax.axis_index("s") * 2 + lax.axis_index("c")
    col0 = pl.multiple_of(wid * _BLK, _BLK)
    ivec = lax.iota(jnp.int32, _L)

    def gather_into_slot(jj, slot):
        return pltpu.async_copy(
            table_hbm.at[half_v.at[jj]],
            rows_big.at[pl.ds(pl.multiple_of(slot * _BLK, _BLK), _BLK)],
            gsem,
        )

    def drain_gather():
        pltpu.make_async_copy(
            table_hbm.at[half_v.at[0]],
            rows_big.at[pl.ds(0, _BLK)],
            gsem,
        ).wait()

    def transpose_block(jj, slot, ot_v):
        parvs = [par_v[jj, pl.ds(i * _L, _L)] for i in range(_BLK // _L)]
        rbase = pl.multiple_of(slot * _BLK, _BLK)
        rrows = [ivec + i * _L + rbase for i in range(_BLK // _L)]

        for d in range(_D):
            vs = [
                plsc.load_gather(rows_big, [rrows[i], parvs[i] + d])
                for i in range(_BLK // _L)
            ]
            for i in range(_BLK // _L):
                ot_v[d, pl.ds(i * _L, _L)] = vs[i]

    def store_block(j0, jj, ot_v, osem):
        row = pl.multiple_of((j0 + jj) * _D, _D)
        return pltpu.async_copy(
            ot_v, out_hbm.at[pl.ds(row, _D), pl.ds(col0, _BLK)], osem)

    def drain_store(osem):
        pltpu.make_async_copy(
            ot0, out_hbm.at[pl.ds(0, _D), pl.ds(col0, _BLK)], osem).wait()

    def step(ja, carry):
        j0 = pl.multiple_of(ja * _JB, _JB)
        pltpu.sync_copy(xt_hbm.at[pl.ds(j0, _JB), pl.ds(col0, _BLK)], idx_v)
        for jj in range(_JB):
            for i in range(_BLK // _L):
                v = idx_v[jj, pl.ds(i * _L, _L)]
                half_v[jj, pl.ds(i * _L, _L)] = lax.shift_right_logical(v, 1)
                par_v[jj, pl.ds(i * _L, _L)] = lax.shift_left(
                    lax.bitwise_and(v, 1), 6)
        for jj in range(4):
            gather_into_slot(jj, jj)

        def pair_body(p, carry2):
            jj = p * 2
            drain_gather()
            transpose_block(jj, jj % 4, ot0)

            @pl.when(p > 0)
            def _():
                drain_store(o0)
            store_block(j0, jj, ot0, o0)

            @pl.when(p < 2)
            def _():
                gather_into_slot(jj + 4, jj % 4)
            drain_gather()
            transpose_block(jj + 1, (jj + 1) % 4, ot1)

            @pl.when(p > 0)
            def _():
                drain_store(o1)
            store_block(j0, jj + 1, ot1, o1)

            @pl.when(p < 2)
            def _():
                gather_into_slot(jj + 5, (jj + 1) % 4)
            return carry2

        lax.fori_loop(0, _JB // 2, pair_body, 0)
        drain_store(o0)
        drain_store(o1)
        return carry

    lax.fori_loop(0, _NJB, step, 0)


def kernel(x, table):
    tt = table.T
    tail_t = table[_NFULL * _CV:].T
    tr = _relayout(tt, tail_t)
    xt = x.T
    out = _embed_gather(xt, tr)
    return out.reshape(_J, _D, _BB).transpose(2, 0, 1)


# speedup vs baseline: 1.3735x; 1.0029x over previous
import functools

import jax
import jax.numpy as jnp
from jax import lax
from jax.experimental import pallas as pl
from jax.experimental.pallas import tpu as pltpu
from jax.experimental.pallas import tpu_sc as plsc

_V = 1_000_000
_D = 64
_J = 200
_BB = 4096
_NW = 32
_BLK = _BB // _NW
_JB = 8
_NJB = _J // _JB
_L = 16

_CV = 256
_NFULL = (_V // 128 // 2) * 2 // 2 * 2 * 64 // 64
_NFULL = 999_936 // _CV
_TAILV = _V - _NFULL * _CV
_PAIRS_PER_IT = 2 * _NW
_NIT = -(-_NFULL // _PAIRS_PER_IT)


@functools.partial(
    pl.kernel,
    out_type=jax.ShapeDtypeStruct((_V // 2, 128), jnp.float32),
    mesh=plsc.VectorSubcoreMesh(core_axis_name="c", subcore_axis_name="s"),
    compiler_params=pltpu.CompilerParams(needs_layout_passes=False),
    scratch_types=[
        pltpu.VMEM((_D, _CV), jnp.float32),
        pltpu.VMEM((_D, _CV), jnp.float32),
        pltpu.VMEM((_CV // 2, 129), jnp.float32),
        pltpu.VMEM((_CV // 2, 129), jnp.float32),
        pltpu.VMEM((_D, _TAILV), jnp.float32),
        pltpu.SemaphoreType.DMA,
        pltpu.SemaphoreType.DMA,
    ],
)
def _relayout(tt_hbm, tail_hbm, out_hbm, in0, in1, out0, out1, tin, sem0, sem1):
    wid = lax.axis_index("s") * 2 + lax.axis_index("c")
    ivec = lax.iota(jnp.int32, _L)
    rowhalf = lax.shift_right_logical(ivec, 1)
    colbase = lax.shift_left(lax.bitwise_and(ivec, 1), 6)

    def transpose_chunk(in_v, out_v):
        def c16_body(c16, carry):
            rbase = rowhalf + c16 * (_L // 2)
            for dg in range(_D // 8):
                vs = [in_v[dg * 8 + t, pl.ds(c16 * _L, _L)] for t in range(8)]
                for t in range(8):
                    plsc.store_scatter(
                        out_v, [rbase, colbase + (dg * 8 + t)], vs[t])
            return carry

        lax.fori_loop(0, _CV // _L, c16_body, 0)

    def body(t, carry):
        g0 = wid + _PAIRS_PER_IT * t
        g1 = wid + _NW + _PAIRS_PER_IT * t
        c0 = jnp.where(g0 < _NFULL, g0 * _CV, 0)
        c1 = jnp.where(g1 < _NFULL, g1 * _CV, 0)
        c0 = pl.multiple_of(c0, _CV)
        c1 = pl.multiple_of(c1, _CV)
        h0 = pltpu.async_copy(tt_hbm.at[:, pl.ds(c0, _CV)], in0, sem0)
        h1 = pltpu.async_copy(tt_hbm.at[:, pl.ds(c1, _CV)], in1, sem1)
        h0.wait()
        transpose_chunk(in0, out0)
        s0 = pltpu.async_copy(
            out0.at[:, pl.ds(0, 128)],
            out_hbm.at[pl.ds(pl.multiple_of(c0 // 2, _CV // 2), _CV // 2)],
            sem0)
        h1.wait()
        transpose_chunk(in1, out1)
        s1 = pltpu.async_copy(
            out1.at[:, pl.ds(0, 128)],
            out_hbm.at[pl.ds(pl.multiple_of(c1 // 2, _CV // 2), _CV // 2)],
            sem1)
        s0.wait()
        s1.wait()
        return carry

    lax.fori_loop(0, _NIT, body, 0)

    @pl.when(wid == 0)
    def _tail():
        pltpu.sync_copy(tail_hbm, tin)
        def c16_body(c16, carry):
            rbase = rowhalf + c16 * (_L // 2)
            for d in range(_D):
                v = tin[d, pl.ds(c16 * _L, _L)]
                plsc.store_scatter(out0, [rbase, colbase + d], v)
            return carry
        lax.fori_loop(0, _TAILV // _L, c16_body, 0)
        pltpu.sync_copy(
            out0.at[pl.ds(0, _TAILV // 2), pl.ds(0, 128)],
            out_hbm.at[pl.ds(_NFULL * _CV // 2, _TAILV // 2)],
        )


@functools.partial(
    pl.kernel,
    out_type=jax.ShapeDtypeStruct((_J * _D, _BB), jnp.float32),
    mesh=plsc.VectorSubcoreMesh(core_axis_name="c", subcore_axis_name="s"),
    compiler_params=pltpu.CompilerParams(needs_layout_passes=False),
    scratch_types=[
        pltpu.VMEM((_JB, _BLK), jnp.int32),
        pltpu.VMEM((_JB, _BLK), jnp.int32),
        pltpu.VMEM((_JB, _BLK), jnp.int32),
        pltpu.VMEM((4 * _BLK, 128), jnp.float32),
        pltpu.VMEM((_D, _BLK), jnp.float32),
        pltpu.VMEM((_D, _BLK), jnp.float32),
        pltpu.SemaphoreType.DMA,
        pltpu.SemaphoreType.DMA,
        pltpu.SemaphoreType.DMA,
    ],
)
def _embed_gather(xt_hbm, table_hbm, out_hbm, idx_v, half_v, par_v,
                  rows_big, ot0, ot1, gsem, o0, o1):
    wid = lax.axis_index("s") * 2 + lax.axis_index("c")
    col0 = pl.multiple_of(wid * _BLK, _BLK)
    ivec = lax.iota(jnp.int32, _L)

    def gather_into_slot(jj, slot):
        return pltpu.async_copy(
            table_hbm.at[half_v.at[jj]],
            rows_big.at[
                pl.ds(pl.multiple_of(slot * _BLK, _BLK), _BLK)
            ],
            gsem,
        )

    def drain_gather():
        pltpu.make_async_copy(
            table_hbm.at[half_v.at[0]],
            rows_big.at[pl.ds(0, _BLK)],
            gsem,
        ).wait()

    def transpose_block(jj, slot, ot_v):
        parvs = [par_v[jj, pl.ds(i * _L, _L)] for i in range(_BLK // _L)]
        rbase = pl.multiple_of(slot * _BLK, _BLK)
        rrows = [ivec + i * _L + rbase for i in range(_BLK // _L)]

        for d in range(_D):
            vs = [
                plsc.load_gather(rows_big, [rrows[i], parvs[i] + d])
                for i in range(_BLK // _L)
            ]
            for i in range(_BLK // _L):
                ot_v[d, pl.ds(i * _L, _L)] = vs[i]

    def store_block(j0, jj, ot_v, osem):
        row = pl.multiple_of((j0 + jj) * _D, _D)
        return pltpu.async_copy(
            ot_v, out_hbm.at[pl.ds(row, _D), pl.ds(col0, _BLK)], osem)

    def drain_store(osem):
        pltpu.make_async_copy(
            ot0, out_hbm.at[pl.ds(0, _D), pl.ds(col0, _BLK)], osem).wait()

    def step(ja, carry):
        j0 = pl.multiple_of(ja * _JB, _JB)
        pltpu.sync_copy(xt_hbm.at[pl.ds(j0, _JB), pl.ds(col0, _BLK)], idx_v)
        for jj in range(_JB):
            for i in range(_BLK // _L):
                v = idx_v[jj, pl.ds(i * _L, _L)]
                half_v[jj, pl.ds(i * _L, _L)] = lax.shift_right_logical(v, 1)
                par_v[jj, pl.ds(i * _L, _L)] = lax.shift_left(
                    lax.bitwise_and(v, 1), 6)
        for jj in range(4):
            gather_into_slot(jj, jj)

        def pair_body(p, carry2):
            jj = p * 2
            drain_gather()
            transpose_block(jj, jj % 4, ot0)

            @pl.when(p > 0)
            def _():
                drain_store(o0)
            store_block(j0, jj, ot0, o0)

            @pl.when(p < 2)
            def _():
                gather_into_slot(jj + 4, jj % 4)
            drain_gather()
            transpose_block(jj + 1, (jj + 1) % 4, ot1)

            @pl.when(p > 0)
            def _():
                drain_store(o1)
            store_block(j0, jj + 1, ot1, o1)

            @pl.when(p < 2)
            def _():
                gather_into_slot(jj + 5, (jj + 1) % 4)
            return carry2

        lax.fori_loop(0, _JB // 2, pair_body, 0)
        drain_store(o0)
        drain_store(o1)
        return carry

    lax.fori_loop(0, _NJB, step, 0)


def kernel(x, table):
    tt = table.T
    tail_t = table[_NFULL * _CV:].T
    tr = _relayout(tt, tail_t)
    xt = x.T
    out = _embed_gather(xt, tr)
    return out.reshape(_J, _D, _BB).transpose(2, 0, 1)


# speedup vs baseline: 2.3481x; 1.7096x over previous
import functools

import jax
import jax.numpy as jnp
from jax import lax
from jax.experimental import pallas as pl
from jax.experimental.pallas import tpu as pltpu
from jax.experimental.pallas import tpu_sc as plsc

_N_VOCAB = 1_000_000
_D = 64
_B = 4096 * 200

_NC = 2
_NS = 16
_NW = _NC * _NS
_B_PER_W = _B // _NW
_G = 128
_NG = 8
_CH = _G * _NG
_N_CHUNKS = _B_PER_W // _CH


@functools.partial(
    pl.kernel,
    out_type=jax.ShapeDtypeStruct((_B, _D), jnp.float32),
    mesh=plsc.VectorSubcoreMesh(core_axis_name="c", subcore_axis_name="s"),
    compiler_params=pltpu.CompilerParams(use_tc_tiling_on_sc=False),
    scratch_types=[
        pltpu.VMEM((_NG, _G), jnp.int32),
        pltpu.VMEM((_CH, _D), jnp.float32),
        pltpu.SemaphoreType.DMA,
    ],
)
def _embed_gather(x_hbm, table_hbm, out_hbm, idx_v, rows_v, sem):
    wid = lax.axis_index("s") * _NC + lax.axis_index("c")
    base = wid * _B_PER_W

    def chunk_body(i, carry):
        off = pl.multiple_of(base + i * _CH, _CH)
        pltpu.sync_copy(
            x_hbm.at[pl.ds(pl.multiple_of(off // _G, _NG), _NG)], idx_v)
        copies = []
        for j in range(_NG):
            copies.append(
                pltpu.async_copy(
                    table_hbm.at[idx_v.at[j]],
                    rows_v.at[pl.ds(j * _G, _G)],
                    sem,
                )
            )
        for c in copies:
            c.wait()
        pltpu.sync_copy(rows_v, out_hbm.at[pl.ds(off, _CH)])
        return carry

    lax.fori_loop(0, _N_CHUNKS, chunk_body, 0)


def kernel(x, table):
    out = _embed_gather(x.reshape(_B // _G, _G), table)
    return out.reshape(x.shape[0], x.shape[1], _D)
